# baseline (speedup 1.0000x reference)
import functools

import jax
import jax.numpy as jnp
from jax import lax
from jax.experimental import pallas as pl
from jax.experimental.pallas import tpu as pltpu
from jax.experimental.pallas import tpu_sc as plsc

_N = 10000
_E = 320000
_F = 64
_NCORE = 2
_NSUB = 16
_NW = _NCORE * _NSUB
_EPW = _E // _NW
_C = 80
_NCH = _EPW // _C
_K = 5
_RPT = 624
_TAIL = _N - _RPT * _NSUB

_BLK = 2000


def _make_prop(nrow, nch):
    rpt = (nrow // _NSUB) // 8 * 8
    tail = nrow - rpt * _NSUB
    mesh = plsc.VectorSubcoreMesh(core_axis_name="c", subcore_axis_name="s")

    @functools.partial(
        pl.kernel,
        out_type=jax.ShapeDtypeStruct((2, nrow, _F), jnp.bfloat16),
        mesh=mesh,
        scratch_types=[
            pltpu.VMEM((nch, _C), jnp.int32),
            pltpu.VMEM((nch, _C), jnp.int32),
            [pltpu.VMEM((_C, _F), jnp.bfloat16) for _ in range(_K)],
            pltpu.VMEM_SHARED((nrow, _F), jnp.bfloat16),
            [pltpu.SemaphoreType.DMA for _ in range(_K)],
        ],
        compiler_params=pltpu.CompilerParams(use_tc_tiling_on_sc=False),
    )
    def prop(h_hbm, src_hbm, dst_hbm, zeros_hbm, out_hbm,
             src_v, dst_v, bufs, acc, sems):
        cid = lax.axis_index("c")
        sid = lax.axis_index("s")
        wid = sid * _NCORE + cid

        def start(chunk, b):
            pltpu.async_copy(h_hbm.at[src_v.at[chunk]], bufs[b], sems[b])

        def wait(b):
            pltpu.make_async_copy(h_hbm.at[src_v.at[0]], bufs[b],
                                  sems[b]).wait()

        pltpu.sync_copy(zeros_hbm.at[pl.ds(sid * rpt, rpt)],
                        acc.at[pl.ds(sid * rpt, rpt)])

        @pl.when(sid == _NSUB - 1)
        def _zero_tail():
            pltpu.sync_copy(zeros_hbm.at[pl.ds(rpt * _NSUB, tail)],
                            acc.at[pl.ds(rpt * _NSUB, tail)])
        pltpu.sync_copy(src_hbm.at[wid], src_v)
        pltpu.sync_copy(dst_hbm.at[wid], dst_v)
        for k in range(_K - 1):
            start(k, k)
        plsc.subcore_barrier()

        def body(i, carry):
            base = i * _K
            for k in range(_K):
                j = base + k
                nxt = jnp.minimum(j + _K - 1, nch - 1)
                start(nxt, (k - 1) % _K)
                wait(k)
                pltpu.sync_copy(bufs[k], acc.at[dst_v.at[j]], add=True)
            return carry

        lax.fori_loop(0, nch // _K, body, 0)
        for k in range(_K - 1):
            wait(k)
        plsc.subcore_barrier()
        pltpu.sync_copy(acc.at[pl.ds(sid * rpt, rpt)],
                        out_hbm.at[cid, pl.ds(sid * rpt, rpt)])

        @pl.when(sid == _NSUB - 1)
        def _write_tail():
            pltpu.sync_copy(acc.at[pl.ds(rpt * _NSUB, tail)],
                            out_hbm.at[cid, pl.ds(rpt * _NSUB, tail)])

    return prop


_prop = _make_prop(_N, _NCH)
_prop2 = _make_prop(2 * _N, 2 * _NCH)


def _row(F):
    return pl.BlockSpec((_BLK, F), lambda i: (i, 0))


def _pair():
    return pl.BlockSpec((2, _BLK, 64), lambda i: (0, i, 0))


def _whole2(a, b):
    return pl.BlockSpec((a, b), lambda i: (0, 0))


_f32 = jax.ShapeDtypeStruct((_N, 64), jnp.float32)
_b16 = jax.ShapeDtypeStruct((_N, 64), jnp.bfloat16)


def _psum(pr):
    return pr[0].astype(jnp.float32) + pr[1].astype(jnp.float32)


def _dense_in(features, W1, b1):
    def body(f, w, b, o, ob):
        y = jnp.maximum(
            jnp.dot(f[...], w[...], preferred_element_type=jnp.float32)
            + b[...], 0.0)
        o[...] = y
        ob[...] = y.astype(jnp.bfloat16)
    return pl.pallas_call(
        body, grid=(_N // _BLK,),
        in_specs=[_row(128), _whole2(128, 64), _whole2(1, 64)],
        out_specs=[_row(64), _row(64)],
        out_shape=[_f32, _b16],
    )(features, W1, b1.reshape(1, 64))


def _cmb1(p, W, b):
    def body(pr, w, bb, o, ob):
        y = (jnp.dot(_psum(pr), w[...], preferred_element_type=jnp.float32)
             + bb[...])
        o[...] = y
        ob[...] = y.astype(jnp.bfloat16)
    return pl.pallas_call(
        body, grid=(_N // _BLK,),
        in_specs=[_pair(), _whole2(64, 64), _whole2(1, 64)],
        out_specs=[_row(64), _row(64)],
        out_shape=[_f32, _b16],
    )(p, W, b.reshape(1, 64))


def _cmb2(p, W, b):
    def body(pr, w, bb, so, yo, ybo):
        s = _psum(pr)
        so[...] = s
        y = jnp.dot(s, w[...], preferred_element_type=jnp.float32) + bb[...]
        yo[...] = y
        ybo[...] = y.astype(jnp.bfloat16)
    return pl.pallas_call(
        body, grid=(_N // _BLK,),
        in_specs=[_pair(), _whole2(64, 64), _whole2(1, 64)],
        out_specs=[_row(64), _row(64), _row(64)],
        out_shape=[_f32, _f32, _b16],
    )(p, W, b.reshape(1, 64))


def _cmb3(s1, p2, Wa, Wb, b):
    def body(s1r, pr, wa, wb, bb, oa, ob, oab, obb):
        y = (jnp.dot(s1r[...], wa[...], preferred_element_type=jnp.float32)
             + jnp.dot(_psum(pr), wb[...], preferred_element_type=jnp.float32)
             + bb[...])
        oa[...] = y[:, :64]
        ob[...] = y[:, 64:]
        oab[...] = y[:, :64].astype(jnp.bfloat16)
        obb[...] = y[:, 64:].astype(jnp.bfloat16)
    return pl.pallas_call(
        body, grid=(_N // _BLK,),
        in_specs=[_row(64), _pair(),
                  _whole2(64, 128), _whole2(64, 128), _whole2(1, 128)],
        out_specs=[_row(64), _row(64), _row(64), _row(64)],
        out_shape=[_f32, _f32, _b16, _b16],
    )(s1, p2, Wa, Wb, b.reshape(1, 128))


def _final(x, x11, x12, x21a, x21b, pm, Wc2, bc2, W2, b2):
    wc2a, wc2b = Wc2[:64], Wc2[64:]
    w2x, w2a, w2b = W2[0:64], W2[64:128], W2[128:192]
    w2ca, w2cb, w2d = W2[192:256], W2[256:320], W2[320:448]

    def dot(a, b):
        return jnp.dot(a, b, preferred_element_type=jnp.float32)

    def body(xr, ar, br, car, cbr, p3r, p4r, wca, wcb, bcc,
             wx, wa, wb, wc1r, wc2r, wd, b2r, o):
        x22 = (dot(_psum(p3r), wca[...])
               + dot(_psum(p4r), wcb[...]) + bcc[...])
        acc = dot(xr[...], wx[...])
        acc = acc + dot(ar[...], wa[...])
        acc = acc + dot(br[...], wb[...])
        acc = acc + dot(car[...], wc1r[...])
        acc = acc + dot(cbr[...], wc2r[...])
        acc = acc + dot(x22, wd[...])
        o[...] = jax.nn.sigmoid(acc + b2r[...])

    nblk = _N // _BLK
    return pl.pallas_call(
        body, grid=(nblk,),
        in_specs=[_row(64), _row(64), _row(64), _row(64), _row(64),
                  pl.BlockSpec((2, _BLK, 64), lambda i: (0, i, 0)),
                  pl.BlockSpec((2, _BLK, 64), lambda i: (0, nblk + i, 0)),
                  _whole2(64, 128), _whole2(64, 128), _whole2(1, 128),
                  _whole2(64, 32), _whole2(64, 32), _whole2(64, 32),
                  _whole2(64, 32), _whole2(64, 32), _whole2(128, 32),
                  _whole2(1, 32)],
        out_specs=_row(32),
        out_shape=jax.ShapeDtypeStruct((_N, 32), jnp.float32),
    )(x, x11, x12, x21a, x21b, pm, pm, wc2a, wc2b, bc2.reshape(1, 128),
      w2x, w2a, w2b, w2ca, w2cb, w2d, b2.reshape(1, 32))


def kernel(features, edge_index, W1, b1, Wc1, bc1, Wc2, bc2, W2, b2):
    src = edge_index[0].reshape(_NW, _NCH, _C)
    dst = edge_index[1].reshape(_NW, _NCH, _C)
    src2 = jnp.concatenate([src, src + _N], axis=1)
    dst2 = jnp.concatenate([dst, dst + _N], axis=1)
    z64 = jnp.zeros((_N, 64), jnp.bfloat16)
    z2 = jnp.zeros((2 * _N, 64), jnp.bfloat16)

    x, xb = _dense_in(features, W1, b1)
    p0 = _prop(xb, src, dst, z64)
    x11, x11b = _cmb1(p0, Wc1, bc1)
    p1 = _prop(x11b, src, dst, z64)
    s1, x12, x12b = _cmb2(p1, Wc1, bc1)
    p2 = _prop(x12b, src, dst, z64)
    x21a, x21b, x21ab, x21bb = _cmb3(s1, p2, Wc2[:64], Wc2[64:], bc2)
    tab = jnp.concatenate([x21ab, x21bb], axis=0)
    pm = _prop2(tab, src2, dst2, z2)
    return _final(x, x11, x12, x21a, x21b, pm, Wc2, bc2, W2, b2)


# speedup vs baseline: 1.1041x; 1.1041x over previous
import functools

import jax
import jax.numpy as jnp
from jax import lax
from jax.experimental import pallas as pl
from jax.experimental.pallas import tpu as pltpu
from jax.experimental.pallas import tpu_sc as plsc

_N = 10000
_E = 320000
_F = 64
_NCORE = 2
_NSUB = 16
_NW = _NCORE * _NSUB
_EPW = _E // _NW
_C = 80
_NCH = _EPW // _C
_K = 5
_RPT = 624
_TAIL = _N - _RPT * _NSUB

_BLK = 2000


def _make_prop(nrow, nch):
    rpt = (nrow // _NSUB) // 8 * 8
    tail = nrow - rpt * _NSUB
    mesh = plsc.VectorSubcoreMesh(core_axis_name="c", subcore_axis_name="s")

    @functools.partial(
        pl.kernel,
        out_type=jax.ShapeDtypeStruct((2, nrow, _F), jnp.bfloat16),
        mesh=mesh,
        scratch_types=[
            pltpu.VMEM((nch, _C), jnp.int32),
            pltpu.VMEM((nch, _C), jnp.int32),
            [pltpu.VMEM((_C, _F), jnp.bfloat16) for _ in range(_K)],
            pltpu.VMEM_SHARED((nrow, _F), jnp.bfloat16),
            [pltpu.SemaphoreType.DMA for _ in range(_K)],
            pltpu.SemaphoreType.DMA,
            pltpu.SemaphoreType.DMA,
        ],
        compiler_params=pltpu.CompilerParams(use_tc_tiling_on_sc=False),
    )
    def prop(h_hbm, src_hbm, dst_hbm, zeros_hbm, out_hbm,
             src_v, dst_v, bufs, acc, sems, semz, semi):
        cid = lax.axis_index("c")
        sid = lax.axis_index("s")
        wid = sid * _NCORE + cid

        def start(chunk, b):
            pltpu.async_copy(h_hbm.at[src_v.at[chunk]], bufs[b], sems[b])

        def wait(b):
            pltpu.make_async_copy(h_hbm.at[src_v.at[0]], bufs[b],
                                  sems[b]).wait()

        czero = pltpu.async_copy(zeros_hbm.at[pl.ds(sid * rpt, rpt)],
                                 acc.at[pl.ds(sid * rpt, rpt)], semz)

        @pl.when(sid == _NSUB - 1)
        def _zero_tail():
            pltpu.sync_copy(zeros_hbm.at[pl.ds(rpt * _NSUB, tail)],
                            acc.at[pl.ds(rpt * _NSUB, tail)])
        csrc = pltpu.async_copy(src_hbm.at[wid], src_v, semi)
        cdst = pltpu.async_copy(dst_hbm.at[wid], dst_v, semi)
        csrc.wait()
        cdst.wait()
        for k in range(_K - 1):
            start(k, k)
        czero.wait()
        plsc.subcore_barrier()

        def body(i, carry):
            base = i * _K
            for k in range(_K):
                j = base + k
                nxt = jnp.minimum(j + _K - 1, nch - 1)
                start(nxt, (k - 1) % _K)
                wait(k)
                pltpu.sync_copy(bufs[k], acc.at[dst_v.at[j]], add=True)
            return carry

        lax.fori_loop(0, nch // _K, body, 0)
        for k in range(_K - 1):
            wait(k)
        plsc.subcore_barrier()
        pltpu.sync_copy(acc.at[pl.ds(sid * rpt, rpt)],
                        out_hbm.at[cid, pl.ds(sid * rpt, rpt)])

        @pl.when(sid == _NSUB - 1)
        def _write_tail():
            pltpu.sync_copy(acc.at[pl.ds(rpt * _NSUB, tail)],
                            out_hbm.at[cid, pl.ds(rpt * _NSUB, tail)])

    return prop


_prop = _make_prop(_N, _NCH)


def _row(F):
    return pl.BlockSpec((_BLK, F), lambda i: (i, 0))


def _pair():
    return pl.BlockSpec((2, _BLK, 64), lambda i: (0, i, 0))


def _whole2(a, b):
    return pl.BlockSpec((a, b), lambda i: (0, 0))


_f32 = jax.ShapeDtypeStruct((_N, 64), jnp.float32)
_b16 = jax.ShapeDtypeStruct((_N, 64), jnp.bfloat16)


def _psum(pr):
    return pr[0].astype(jnp.float32) + pr[1].astype(jnp.float32)


def _dense_in(features, W1, b1):
    def body(f, w, b, o, ob):
        y = jnp.maximum(
            jnp.dot(f[...], w[...], preferred_element_type=jnp.float32)
            + b[...], 0.0)
        o[...] = y
        ob[...] = y.astype(jnp.bfloat16)
    return pl.pallas_call(
        body, grid=(_N // _BLK,),
        in_specs=[_row(128), _whole2(128, 64), _whole2(1, 64)],
        out_specs=[_row(64), _row(64)],
        out_shape=[_f32, _b16],
    )(features, W1, b1.reshape(1, 64))


def _cmb1(p, W, b):
    def body(pr, w, bb, o, ob):
        y = (jnp.dot(_psum(pr), w[...], preferred_element_type=jnp.float32)
             + bb[...])
        o[...] = y
        ob[...] = y.astype(jnp.bfloat16)
    return pl.pallas_call(
        body, grid=(_N // _BLK,),
        in_specs=[_pair(), _whole2(64, 64), _whole2(1, 64)],
        out_specs=[_row(64), _row(64)],
        out_shape=[_f32, _b16],
    )(p, W, b.reshape(1, 64))


def _cmb2(p, W, b):
    def body(pr, w, bb, so, yo, ybo):
        s = _psum(pr)
        so[...] = s
        y = jnp.dot(s, w[...], preferred_element_type=jnp.float32) + bb[...]
        yo[...] = y
        ybo[...] = y.astype(jnp.bfloat16)
    return pl.pallas_call(
        body, grid=(_N // _BLK,),
        in_specs=[_pair(), _whole2(64, 64), _whole2(1, 64)],
        out_specs=[_row(64), _row(64), _row(64)],
        out_shape=[_f32, _f32, _b16],
    )(p, W, b.reshape(1, 64))


def _cmb3(s1, p2, Wa, Wb, b):
    def body(s1r, pr, wa, wb, bb, oa, ob, oab, obb):
        y = (jnp.dot(s1r[...], wa[...], preferred_element_type=jnp.float32)
             + jnp.dot(_psum(pr), wb[...], preferred_element_type=jnp.float32)
             + bb[...])
        oa[...] = y[:, :64]
        ob[...] = y[:, 64:]
        oab[...] = y[:, :64].astype(jnp.bfloat16)
        obb[...] = y[:, 64:].astype(jnp.bfloat16)
    return pl.pallas_call(
        body, grid=(_N // _BLK,),
        in_specs=[_row(64), _pair(),
                  _whole2(64, 128), _whole2(64, 128), _whole2(1, 128)],
        out_specs=[_row(64), _row(64), _row(64), _row(64)],
        out_shape=[_f32, _f32, _b16, _b16],
    )(s1, p2, Wa, Wb, b.reshape(1, 128))


def _final(x, x11, x12, x21a, x21b, p3, p4, Wc2, bc2, W2, b2):
    wc2a, wc2b = Wc2[:64], Wc2[64:]
    w2x, w2a, w2b = W2[0:64], W2[64:128], W2[128:192]
    w2ca, w2cb, w2d = W2[192:256], W2[256:320], W2[320:448]

    def dot(a, b):
        return jnp.dot(a, b, preferred_element_type=jnp.float32)

    def body(xr, ar, br, car, cbr, p3r, p4r, wca, wcb, bcc,
             wx, wa, wb, wc1r, wc2r, wd, b2r, o):
        x22 = (dot(_psum(p3r), wca[...])
               + dot(_psum(p4r), wcb[...]) + bcc[...])
        acc = dot(xr[...], wx[...])
        acc = acc + dot(ar[...], wa[...])
        acc = acc + dot(br[...], wb[...])
        acc = acc + dot(car[...], wc1r[...])
        acc = acc + dot(cbr[...], wc2r[...])
        acc = acc + dot(x22, wd[...])
        o[...] = jax.nn.sigmoid(acc + b2r[...])

    return pl.pallas_call(
        body, grid=(_N // _BLK,),
        in_specs=[_row(64), _row(64), _row(64), _row(64), _row(64),
                  _pair(), _pair(),
                  _whole2(64, 128), _whole2(64, 128), _whole2(1, 128),
                  _whole2(64, 32), _whole2(64, 32), _whole2(64, 32),
                  _whole2(64, 32), _whole2(64, 32), _whole2(128, 32),
                  _whole2(1, 32)],
        out_specs=_row(32),
        out_shape=jax.ShapeDtypeStruct((_N, 32), jnp.float32),
    )(x, x11, x12, x21a, x21b, p3, p4, wc2a, wc2b, bc2.reshape(1, 128),
      w2x, w2a, w2b, w2ca, w2cb, w2d, b2.reshape(1, 32))


def kernel(features, edge_index, W1, b1, Wc1, bc1, Wc2, bc2, W2, b2):
    src = edge_index[0].reshape(_NW, _NCH, _C)
    dst = edge_index[1].reshape(_NW, _NCH, _C)
    z64 = jnp.zeros((_N, 64), jnp.bfloat16)

    x, xb = _dense_in(features, W1, b1)
    p0 = _prop(xb, src, dst, z64)
    x11, x11b = _cmb1(p0, Wc1, bc1)
    p1 = _prop(x11b, src, dst, z64)
    s1, x12, x12b = _cmb2(p1, Wc1, bc1)
    p2 = _prop(x12b, src, dst, z64)
    x21a, x21b, x21ab, x21bb = _cmb3(s1, p2, Wc2[:64], Wc2[64:], bc2)
    p3 = _prop(x21ab, src, dst, z64)
    p4 = _prop(x21bb, src, dst, z64)
    return _final(x, x11, x12, x21a, x21b, p3, p4, Wc2, bc2, W2, b2)
